# baseline (speedup 1.0000x reference)
import jax
import jax.numpy as jnp
from jax import lax
from jax.experimental import pallas as pl
from jax.experimental.pallas import tpu as pltpu
from jax.experimental.pallas import tpu_sc as plsc

B, T, F = 4096, 200, 64
S = 50
NC, NS, L = 2, 16, 16
NW = NC * NS
LT = 128
GROUPS = LT // L
NP = F // 2


def _body(x_hbm, idx_hbm, out_hbm,
          idx_v, sa0, sb0, sa1, sb1, oa, ob, sg0, sg1, sw):
    j = lax.axis_index("s") * NC + lax.axis_index("c")
    iota = lax.iota(jnp.int32, L)
    lane_vecs = [g * L + iota for g in range(GROUPS)]
    lanes = j * LT

    pltpu.sync_copy(idx_hbm.at[:, pl.ds(lanes, LT)], idx_v)

    def fire_g(p, slab_a, slab_b, sem):
        pltpu.async_copy(x_hbm.at[:, 2 * p, pl.ds(lanes, LT)], slab_a, sem)
        pltpu.async_copy(x_hbm.at[:, 2 * p + 1, pl.ds(lanes, LT)], slab_b, sem)

    def drain_g(slab_a, slab_b, sem):
        pltpu.make_async_copy(
            x_hbm.at[:, 0, pl.ds(lanes, LT)], slab_a, sem).wait()
        pltpu.make_async_copy(
            x_hbm.at[:, 0, pl.ds(lanes, LT)], slab_b, sem).wait()

    def fire_w(p):
        pltpu.async_copy(oa, out_hbm.at[:, 2 * p, pl.ds(lanes, LT)], sw)
        pltpu.async_copy(ob, out_hbm.at[:, 2 * p + 1, pl.ds(lanes, LT)], sw)

    def drain_w():
        pltpu.make_async_copy(
            oa, out_hbm.at[:, 0, pl.ds(lanes, LT)], sw).wait()
        pltpu.make_async_copy(
            ob, out_hbm.at[:, 0, pl.ds(lanes, LT)], sw).wait()

    def compute(slab_a, slab_b):
        def s_body(s, carry):
            for g in range(GROUPS):
                tv = idx_v[s, pl.ds(g * L, L)]
                oa[s, pl.ds(g * L, L)] = plsc.load_gather(
                    slab_a, [tv, lane_vecs[g]])
                ob[s, pl.ds(g * L, L)] = plsc.load_gather(
                    slab_b, [tv, lane_vecs[g]])
            return carry

        lax.fori_loop(0, S, s_body, 0)

    bufs = ((sa0, sb0, sg0), (sa1, sb1, sg1))

    fire_g(0, sa0, sb0, sg0)
    fire_g(1, sa1, sb1, sg1)
    drain_g(sa0, sb0, sg0)
    compute(sa0, sb0)
    fire_w(0)
    fire_g(2, sa0, sb0, sg0)

    def p_body(k, carry):
        for b, (sa, sb, sg) in enumerate(bufs):
            p = 2 * k + b
            drain_g(sa, sb, sg)
            drain_w()
            compute(sa, sb)
            fire_w(p)
            fire_g(p + 2, sa, sb, sg)
        return carry

    drain_g(sa1, sb1, sg1)
    drain_w()
    compute(sa1, sb1)
    fire_w(1)
    fire_g(3, sa1, sb1, sg1)

    lax.fori_loop(1, NP // 2 - 1, p_body, 0)

    for b, (sa, sb, sg) in enumerate(bufs):
        p = NP - 2 + b
        drain_g(sa, sb, sg)
        drain_w()
        compute(sa, sb)
        fire_w(p)
    drain_w()


@jax.jit
def kernel(x, indices):
    xt = jnp.transpose(x, (1, 2, 0))
    idxt = jnp.transpose(indices.astype(jnp.int32), (1, 0))
    call = pl.kernel(
        _body,
        out_type=jax.ShapeDtypeStruct((S, F, B), jnp.float32),
        mesh=plsc.VectorSubcoreMesh(core_axis_name="c", subcore_axis_name="s"),
        scratch_types=[
            pltpu.VMEM((S, LT), jnp.int32),
            pltpu.VMEM((T, LT), jnp.float32),
            pltpu.VMEM((T, LT), jnp.float32),
            pltpu.VMEM((T, LT), jnp.float32),
            pltpu.VMEM((T, LT), jnp.float32),
            pltpu.VMEM((S, LT), jnp.float32),
            pltpu.VMEM((S, LT), jnp.float32),
            pltpu.SemaphoreType.DMA,
            pltpu.SemaphoreType.DMA,
            pltpu.SemaphoreType.DMA,
        ],
        compiler_params=pltpu.CompilerParams(
            use_tc_tiling_on_sc=True, needs_layout_passes=False
        ),
    )
    outt = call(xt, idxt)
    return jnp.transpose(outt, (2, 0, 1))


# speedup vs baseline: 1.0471x; 1.0471x over previous
import jax
import jax.numpy as jnp
from jax import lax
from jax.experimental import pallas as pl
from jax.experimental.pallas import tpu as pltpu
from jax.experimental.pallas import tpu_sc as plsc

B, T, F = 4096, 200, 64
S = 50
NC, NS, L = 2, 16, 16
NW = NC * NS
LT = 128
GROUPS = LT // L
SU = 5


def _body(x_hbm, idx_hbm, out_hbm,
          idx_v, slab0, slab1, out0, out1, sg0, sg1, sw0, sw1):
    j = lax.axis_index("s") * NC + lax.axis_index("c")
    iota = lax.iota(jnp.int32, L)
    lane_vecs = [g * L + iota for g in range(GROUPS)]
    lanes = j * LT

    pltpu.sync_copy(idx_hbm.at[:, pl.ds(lanes, LT)], idx_v)

    def fire_g(fc, slab_ref, sem):
        h = T // 2
        pltpu.async_copy(
            x_hbm.at[pl.ds(0, h), fc, pl.ds(lanes, LT)],
            slab_ref.at[pl.ds(0, h), :], sem,
        )
        pltpu.async_copy(
            x_hbm.at[pl.ds(h, h), fc, pl.ds(lanes, LT)],
            slab_ref.at[pl.ds(h, h), :], sem,
        )

    def drain_g(slab_ref, sem):
        pltpu.make_async_copy(
            x_hbm.at[:, 0, pl.ds(lanes, LT)], slab_ref, sem
        ).wait()

    def fire_w(fc, out_ref, sem):
        pltpu.async_copy(out_ref, out_hbm.at[:, fc, pl.ds(lanes, LT)], sem)

    def drain_w(out_ref, sem):
        pltpu.make_async_copy(
            out_ref, out_hbm.at[:, 0, pl.ds(lanes, LT)], sem
        ).wait()

    def compute(slab_ref, out_ref):
        def s_body(k, carry):
            for i in range(SU):
                s = k * SU + i
                for g in range(GROUPS):
                    tv = idx_v[s, pl.ds(g * L, L)]
                    out_ref[s, pl.ds(g * L, L)] = plsc.load_gather(
                        slab_ref, [tv, lane_vecs[g]]
                    )
            return carry

        lax.fori_loop(0, S // SU, s_body, 0)

    bufs = ((slab0, out0, sg0, sw0), (slab1, out1, sg1, sw1))

    fire_g(0, slab0, sg0)
    fire_g(1, slab1, sg1)
    for fc in (0, 1):
        slab_r, out_r, sg, sw = bufs[fc]
        drain_g(slab_r, sg)
        compute(slab_r, out_r)
        fire_w(fc, out_r, sw)
        fire_g(fc + 2, slab_r, sg)

    def k_body(k, carry):
        for b, (slab_r, out_r, sg, sw) in enumerate(bufs):
            fc = 2 * k + b
            drain_g(slab_r, sg)
            drain_w(out_r, sw)
            compute(slab_r, out_r)
            fire_w(fc, out_r, sw)
            fire_g(fc + 2, slab_r, sg)
        return carry

    lax.fori_loop(1, F // 2 - 1, k_body, 0)

    for b, (slab_r, out_r, sg, sw) in enumerate(bufs):
        fc = F - 2 + b
        drain_g(slab_r, sg)
        drain_w(out_r, sw)
        compute(slab_r, out_r)
        fire_w(fc, out_r, sw)
    for b, (slab_r, out_r, sg, sw) in enumerate(bufs):
        drain_w(out_r, sw)


@jax.jit
def kernel(x, indices):
    xt = jnp.transpose(x, (1, 2, 0))
    idxt = jnp.transpose(indices.astype(jnp.int32), (1, 0))
    call = pl.kernel(
        _body,
        out_type=jax.ShapeDtypeStruct((S, F, B), jnp.float32),
        mesh=plsc.VectorSubcoreMesh(core_axis_name="c", subcore_axis_name="s"),
        scratch_types=[
            pltpu.VMEM((S, LT), jnp.int32),
            pltpu.VMEM((T, LT), jnp.float32),
            pltpu.VMEM((T, LT), jnp.float32),
            pltpu.VMEM((S, LT), jnp.float32),
            pltpu.VMEM((S, LT), jnp.float32),
            pltpu.SemaphoreType.DMA,
            pltpu.SemaphoreType.DMA,
            pltpu.SemaphoreType.DMA,
            pltpu.SemaphoreType.DMA,
        ],
        compiler_params=pltpu.CompilerParams(
            use_tc_tiling_on_sc=True, needs_layout_passes=False
        ),
    )
    outt = call(xt, idxt)
    return jnp.transpose(outt, (2, 0, 1))


# speedup vs baseline: 1.0758x; 1.0274x over previous
import jax
import jax.numpy as jnp
from jax import lax
from jax.experimental import pallas as pl
from jax.experimental.pallas import tpu as pltpu
from jax.experimental.pallas import tpu_sc as plsc

B, T, F = 4096, 200, 64
S = 50
NC, NS, L = 2, 16, 16
NW = NC * NS
LT = 128
GROUPS = LT // L


def _body(x_hbm, idx_hbm, out_hbm,
          idx_v, slab0, slab1, out0, out1, sg0, sg1, sw0, sw1):
    j = lax.axis_index("s") * NC + lax.axis_index("c")
    iota = lax.iota(jnp.int32, L)
    lane_vecs = [g * L + iota for g in range(GROUPS)]
    lanes = j * LT

    pltpu.sync_copy(idx_hbm.at[:, pl.ds(lanes, LT)], idx_v)

    def fire_g(fc, slab_ref, sem):
        h = T // 2
        pltpu.async_copy(
            x_hbm.at[pl.ds(0, h), fc, pl.ds(lanes, LT)],
            slab_ref.at[pl.ds(0, h), :], sem,
        )
        pltpu.async_copy(
            x_hbm.at[pl.ds(h, h), fc, pl.ds(lanes, LT)],
            slab_ref.at[pl.ds(h, h), :], sem,
        )

    def drain_g(slab_ref, sem):
        pltpu.make_async_copy(
            x_hbm.at[:, 0, pl.ds(lanes, LT)], slab_ref, sem
        ).wait()

    def fire_w(fc, out_ref, sem):
        pltpu.async_copy(out_ref, out_hbm.at[:, fc, pl.ds(lanes, LT)], sem)

    def drain_w(out_ref, sem):
        pltpu.make_async_copy(
            out_ref, out_hbm.at[:, 0, pl.ds(lanes, LT)], sem
        ).wait()

    def compute(slab_ref, out_ref):
        def s_body(s, carry):
            for g in range(GROUPS):
                tv = idx_v[s, pl.ds(g * L, L)]
                out_ref[s, pl.ds(g * L, L)] = plsc.load_gather(
                    slab_ref, [tv, lane_vecs[g]]
                )
            return carry

        lax.fori_loop(0, S, s_body, 0)

    bufs = ((slab0, out0, sg0, sw0), (slab1, out1, sg1, sw1))

    fire_g(0, slab0, sg0)
    fire_g(1, slab1, sg1)
    for fc in (0, 1):
        slab_r, out_r, sg, sw = bufs[fc]
        drain_g(slab_r, sg)
        compute(slab_r, out_r)
        fire_w(fc, out_r, sw)
        fire_g(fc + 2, slab_r, sg)

    def k_body(k, carry):
        for b, (slab_r, out_r, sg, sw) in enumerate(bufs):
            fc = 2 * k + b
            drain_g(slab_r, sg)
            drain_w(out_r, sw)
            compute(slab_r, out_r)
            fire_w(fc, out_r, sw)
            fire_g(fc + 2, slab_r, sg)
        return carry

    lax.fori_loop(1, F // 2 - 1, k_body, 0)

    for b, (slab_r, out_r, sg, sw) in enumerate(bufs):
        fc = F - 2 + b
        drain_g(slab_r, sg)
        drain_w(out_r, sw)
        compute(slab_r, out_r)
        fire_w(fc, out_r, sw)
    for b, (slab_r, out_r, sg, sw) in enumerate(bufs):
        drain_w(out_r, sw)


@jax.jit
def kernel(x, indices):
    xt = jnp.transpose(x, (1, 2, 0))
    idxt = jnp.transpose(indices.astype(jnp.int32), (1, 0))
    call = pl.kernel(
        _body,
        out_type=jax.ShapeDtypeStruct((S, F, B), jnp.float32),
        mesh=plsc.VectorSubcoreMesh(core_axis_name="c", subcore_axis_name="s"),
        scratch_types=[
            pltpu.VMEM((S, LT), jnp.int32),
            pltpu.VMEM((T, LT), jnp.float32),
            pltpu.VMEM((T, LT), jnp.float32),
            pltpu.VMEM((S, LT), jnp.float32),
            pltpu.VMEM((S, LT), jnp.float32),
            pltpu.SemaphoreType.DMA,
            pltpu.SemaphoreType.DMA,
            pltpu.SemaphoreType.DMA,
            pltpu.SemaphoreType.DMA,
        ],
        compiler_params=pltpu.CompilerParams(
            use_tc_tiling_on_sc=True, needs_layout_passes=False
        ),
    )
    outt = call(xt, idxt)
    return jnp.transpose(outt, (2, 0, 1))
